# baseline (speedup 1.0000x reference)
import jax
import jax.numpy as jnp
from jax import lax
from jax.experimental import pallas as pl
from jax.experimental.pallas import tpu as pltpu
from jax.experimental.pallas import tpu_sc as plsc
import functools

NC = 2
NS = 16
NW = NC * NS

R = 16384
S = 50
D = 128
B = R * S
B_PER_W = B // NW
NBUF = 2
CH = 400
N_GRP = B_PER_W // (NBUF * CH)
G_OFF = (0, 96, 192, 288)
G_LEN = (96, 96, 96, 112)


def _emb_body(idx_hbm, table_hbm, out_hbm, idx_v,
              rows_0, rows_1,
              gsem_0, gsem_1,
              ssem_0, ssem_1):
    rows = (rows_0, rows_1)
    gsem = (gsem_0, gsem_1)
    ssem = (ssem_0, ssem_1)
    wid = lax.axis_index("s") * NC + lax.axis_index("c")
    base = wid * B_PER_W
    pltpu.sync_copy(idx_hbm.at[pl.ds(base, B_PER_W)], idx_v)

    def drain_scatter(b, pos):
        pltpu.make_async_copy(rows[b], out_hbm.at[pl.ds(pos, CH)],
                              ssem[b]).wait()

    def fire_gathers(b, off):
        return [pltpu.async_copy(table_hbm.at[idx_v.at[pl.ds(off + o, n)]],
                                 rows[b].at[pl.ds(o, n)], gsem[b])
                for o, n in zip(G_OFF, G_LEN)]

    @pl.loop(0, N_GRP)
    def _grp(t):
        pos0 = base + t * (NBUF * CH)
        gs = []
        for b in range(NBUF):
            @pl.when(t > 0)
            def _(b=b):
                drain_scatter(b, pos0 + b * CH)
            gs.append(fire_gathers(b, (t * NBUF + b) * CH))
        for b in range(NBUF):
            for g in gs[b]:
                g.wait()
            pltpu.async_copy(rows[b], out_hbm.at[pl.ds(pos0 + b * CH, CH)],
                             ssem[b])

    for b in range(NBUF):
        drain_scatter(b, base + b * CH)


@functools.partial(jax.jit, static_argnames=())
def _emb_lookup(idx_flat, table):
    mesh = plsc.VectorSubcoreMesh(core_axis_name="c", subcore_axis_name="s")
    f = pl.kernel(
        _emb_body,
        out_type=jax.ShapeDtypeStruct((B, D), jnp.float32),
        mesh=mesh,
        scratch_types=(
            [pltpu.VMEM((B_PER_W,), jnp.int32)]
            + [pltpu.VMEM((CH, D), jnp.float32) for _ in range(NBUF)]
            + [pltpu.SemaphoreType.DMA for _ in range(2 * NBUF)]
        ),
    )
    return f(idx_flat, table)


def kernel(x, table):
    idx_flat = x.T.astype(jnp.int32).reshape(-1)
    out = _emb_lookup(idx_flat, table)
    return out.reshape(S, R, D).transpose(1, 0, 2)


# speedup vs baseline: 1.0245x; 1.0245x over previous
import jax
import jax.numpy as jnp
from jax import lax
from jax.experimental import pallas as pl
from jax.experimental.pallas import tpu as pltpu
from jax.experimental.pallas import tpu_sc as plsc
import functools

NC = 2
NS = 16
NW = NC * NS

R = 16384
S = 50
D = 128
B = R * S
B_PER_W = B // NW
NBUF = 4
CH = 200
N_GRP = B_PER_W // (NBUF * CH)
G_OFF = (0, 96)
G_LEN = (96, 104)


def _emb_body(idx_hbm, table_hbm, out_hbm, idx_v,
              rows_0, rows_1, rows_2, rows_3,
              gsem_0, gsem_1, gsem_2, gsem_3,
              ssem_0, ssem_1, ssem_2, ssem_3):
    rows = (rows_0, rows_1, rows_2, rows_3)
    gsem = (gsem_0, gsem_1, gsem_2, gsem_3)
    ssem = (ssem_0, ssem_1, ssem_2, ssem_3)
    wid = lax.axis_index("s") * NC + lax.axis_index("c")
    base = wid * B_PER_W
    pltpu.sync_copy(idx_hbm.at[pl.ds(base, B_PER_W)], idx_v)

    def drain_scatter(b, pos):
        for o, n in zip(G_OFF, G_LEN):
            pltpu.make_async_copy(rows[b].at[pl.ds(o, n)],
                                  out_hbm.at[pl.ds(pos + o, n)],
                                  ssem[b]).wait()

    def fire_gathers(b, off):
        return [pltpu.async_copy(table_hbm.at[idx_v.at[pl.ds(off + o, n)]],
                                 rows[b].at[pl.ds(o, n)], gsem[b])
                for o, n in zip(G_OFF, G_LEN)]

    @pl.loop(0, N_GRP)
    def _grp(t):
        pos0 = base + t * (NBUF * CH)
        gs = []
        for b in range(NBUF):
            @pl.when(t > 0)
            def _(b=b):
                drain_scatter(b, pos0 + b * CH)
            gs.append(fire_gathers(b, (t * NBUF + b) * CH))
        for b in range(NBUF):
            for (o, n), g in zip(zip(G_OFF, G_LEN), gs[b]):
                g.wait()
                pltpu.async_copy(rows[b].at[pl.ds(o, n)],
                                 out_hbm.at[pl.ds(pos0 + b * CH + o, n)],
                                 ssem[b])

    for b in range(NBUF):
        drain_scatter(b, base + b * CH)


@functools.partial(jax.jit, static_argnames=())
def _emb_lookup(idx_flat, table):
    mesh = plsc.VectorSubcoreMesh(core_axis_name="c", subcore_axis_name="s")
    f = pl.kernel(
        _emb_body,
        out_type=jax.ShapeDtypeStruct((B, D), jnp.float32),
        mesh=mesh,
        scratch_types=(
            [pltpu.VMEM((B_PER_W,), jnp.int32)]
            + [pltpu.VMEM((CH, D), jnp.float32) for _ in range(NBUF)]
            + [pltpu.SemaphoreType.DMA for _ in range(2 * NBUF)]
        ),
    )
    return f(idx_flat, table)


def kernel(x, table):
    idx_flat = x.T.astype(jnp.int32).reshape(-1)
    out = _emb_lookup(idx_flat, table)
    return out.reshape(S, R, D).transpose(1, 0, 2)


# speedup vs baseline: 1.0257x; 1.0012x over previous
import jax
import jax.numpy as jnp
from jax import lax
from jax.experimental import pallas as pl
from jax.experimental.pallas import tpu as pltpu
from jax.experimental.pallas import tpu_sc as plsc
import functools

NC = 2
NS = 16
NW = NC * NS

R = 16384
S = 50
D = 128
B = R * S
B_PER_W = B // NW
NBUF = 4
CH = 200
N_GRP = B_PER_W // (NBUF * CH)
G_OFF = (0, 96)
G_LEN = (96, 104)


def _emb_body(idx_hbm, table_hbm, out_hbm, idx_v,
              rows_0, rows_1, rows_2, rows_3,
              gsem_0, gsem_1, gsem_2, gsem_3,
              ssem_0, ssem_1, ssem_2, ssem_3):
    rows = (rows_0, rows_1, rows_2, rows_3)
    gsem = (gsem_0, gsem_1, gsem_2, gsem_3)
    ssem = (ssem_0, ssem_1, ssem_2, ssem_3)
    wid = lax.axis_index("s") * NC + lax.axis_index("c")
    base = wid * B_PER_W
    pltpu.sync_copy(idx_hbm.at[pl.ds(base, B_PER_W)], idx_v)

    def drain_scatter(b, pos):
        pltpu.make_async_copy(rows[b], out_hbm.at[pl.ds(pos, CH)],
                              ssem[b]).wait()

    def fire_gathers(b, off):
        return [pltpu.async_copy(table_hbm.at[idx_v.at[pl.ds(off + o, n)]],
                                 rows[b].at[pl.ds(o, n)], gsem[b])
                for o, n in zip(G_OFF, G_LEN)]

    @pl.loop(0, N_GRP)
    def _grp(t):
        pos0 = base + t * (NBUF * CH)
        gs = []
        for b in range(NBUF):
            @pl.when(t > 0)
            def _(b=b):
                drain_scatter(b, pos0 + b * CH)
            gs.append(fire_gathers(b, (t * NBUF + b) * CH))
        for b in range(NBUF):
            for g in gs[b]:
                g.wait()
            pltpu.async_copy(rows[b], out_hbm.at[pl.ds(pos0 + b * CH, CH)],
                             ssem[b])

    for b in range(NBUF):
        drain_scatter(b, base + b * CH)


@functools.partial(jax.jit, static_argnames=())
def _emb_lookup(idx_flat, table):
    mesh = plsc.VectorSubcoreMesh(core_axis_name="c", subcore_axis_name="s")
    f = pl.kernel(
        _emb_body,
        out_type=jax.ShapeDtypeStruct((B, D), jnp.float32),
        mesh=mesh,
        scratch_types=(
            [pltpu.VMEM((B_PER_W,), jnp.int32)]
            + [pltpu.VMEM((CH, D), jnp.float32) for _ in range(NBUF)]
            + [pltpu.SemaphoreType.DMA for _ in range(2 * NBUF)]
        ),
    )
    return f(idx_flat, table)


def kernel(x, table):
    idx_flat = x.T.astype(jnp.int32).reshape(-1)
    out = _emb_lookup(idx_flat, table)
    return out.reshape(S, R, D).transpose(1, 0, 2)
